# baseline (speedup 1.0000x reference)
import functools

import jax
import jax.numpy as jnp
from jax import lax
from jax.experimental import pallas as pl
from jax.experimental.pallas import tpu as pltpu
from jax.experimental.pallas import tpu_sc as plsc

N = 10000
H = 128
OUT = 64
R = 3
E = 320000

NC = 2
NS = 16
L = 16

CH = 128
NBUF = 2
S0 = 152
S1 = 8
NCHUNK = NS * (S0 + S1)
EPAD = NCHUNK * CH
IB = 20
QPB = IB // NBUF

RPT = 632
P = NS * RPT
ZR = 16

BLK = 1264
GRID = P // BLK


def _make_edge_pass(width, ntab, with_deg, tab_bf16=False):
    mesh = plsc.VectorSubcoreMesh(core_axis_name="c", subcore_axis_name="s")
    n_deg = R if with_deg else 0
    row_dt = jnp.bfloat16 if tab_bf16 else jnp.float32
    out_type = (
        [jax.ShapeDtypeStruct((NC, P, width), jnp.float32) for _ in range(R)]
        + [jax.ShapeDtypeStruct((NC, P), jnp.float32) for _ in range(n_deg)]
    )
    scratch = (
        [pltpu.VMEM_SHARED((P, width), jnp.float32),
         pltpu.VMEM((IB, CH), jnp.int32),
         pltpu.VMEM((IB, CH), jnp.int32)]
        + [pltpu.VMEM((CH, width), row_dt) for _ in range(NBUF)]
        + ([pltpu.VMEM((CH, width), jnp.float32)] if tab_bf16 else [])
        + [pltpu.VMEM((ZR, width), jnp.float32)]
        + [pltpu.SemaphoreType.DMA for _ in range(NBUF)]
    )
    if with_deg:
        scratch.insert(1, pltpu.VMEM_SHARED((P,), jnp.float32))
        scratch.append(pltpu.VMEM((CH,), jnp.float32))
        scratch.append(pltpu.VMEM((CH,), jnp.float32))

    @functools.partial(
        pl.kernel, out_type=out_type, scratch_types=scratch, mesh=mesh,
        name="edge_pass",
        compiler_params=pltpu.CompilerParams(
            use_tc_tiling_on_sc=False,
            needs_layout_passes=not tab_bf16))
    def run(*refs):
        i = 0
        tabs = refs[i:i + ntab]; i += ntab
        srcs_hbm, dsts_hbm = refs[i:i + 2]; i += 2
        out_acc = refs[i:i + R]; i += R
        out_deg = refs[i:i + n_deg]; i += n_deg
        acc = refs[i]; i += 1
        if with_deg:
            deg = refs[i]; i += 1
        srcb, dstb = refs[i:i + 2]; i += 2
        rows = refs[i:i + NBUF]; i += NBUF
        if tab_bf16:
            fbuf = refs[i]; i += 1
        zbuf = refs[i]; i += 1
        sems = refs[i:i + NBUF]; i += NBUF
        if with_deg:
            ones_v, z1v = refs[i:i + 2]

        cid = lax.axis_index("c")
        sid = lax.axis_index("s")
        rb = sid * RPT
        cbase = jnp.where(cid == 0, sid * S0, NS * S0 + sid * S1)
        nquads = jnp.where(cid == 0, S0 // NBUF, S1 // NBUF)

        zv = jnp.zeros((L,), jnp.float32)
        for zi in range(ZR):
            for zk in range(width // L):
                zbuf[zi, pl.ds(zk * L, L)] = zv
        if with_deg:
            ov = jnp.full((L,), 1.0, jnp.float32)
            for zk in range(CH // L):
                ones_v[pl.ds(zk * L, L)] = ov
            for zk in range(CH // L):
                z1v[pl.ds(zk * L, L)] = zv

        for r in range(R):
            with jax.named_scope("zero_phase"):
                @pl.loop(0, RPT // ZR)
                def _zero(k):
                    pltpu.sync_copy(zbuf, acc.at[pl.ds(rb + k * ZR, ZR)])
                if RPT % ZR:
                    pltpu.sync_copy(
                        zbuf.at[pl.ds(0, RPT % ZR)],
                        acc.at[pl.ds(rb + RPT - RPT % ZR, RPT % ZR)])
                if with_deg:
                    @pl.when(sid == 0)
                    def _zero_deg():
                        @pl.loop(0, P // CH)
                        def _zd(k):
                            pltpu.sync_copy(z1v, deg.at[pl.ds(k * CH, CH)])
                plsc.subcore_barrier()

            tab = tabs[r % ntab]

            with jax.named_scope("edges_phase"):
                @pl.loop(0, nquads)
                def _quad(q, tab=tab):
                    @pl.when(q % QPB == 0)
                    def _reload():
                        blk = pl.ds(cbase + NBUF * q, IB)
                        pltpu.sync_copy(srcs_hbm.at[r].at[blk], srcb)
                        pltpu.sync_copy(dsts_hbm.at[r].at[blk], dstb)
                    js = [(NBUF * q + b) % IB for b in range(NBUF)]
                    cps = [pltpu.async_copy(tab.at[srcb.at[js[b]]],
                                            rows[b], sems[b])
                           for b in range(NBUF)]
                    if with_deg:
                        for b in range(NBUF):
                            pltpu.sync_copy(ones_v, deg.at[dstb.at[js[b]]],
                                            add=True)
                    for b in range(NBUF):
                        cps[b].wait()
                        if tab_bf16:
                            @pl.loop(0, CH)
                            def _cvt(row, b=b):
                                for g in range(width // 32):
                                    v = rows[b][row, pl.ds(g * 32, 32)]
                                    lo, hi = plsc.unpack(
                                        v, format=plsc.PackFormat.INTERLEAVED)
                                    fbuf[row, pl.ds(g * 32, L)] = lo
                                    fbuf[row, pl.ds(g * 32 + L, L)] = hi
                            pltpu.sync_copy(fbuf, acc.at[dstb.at[js[b]]],
                                            add=True)
                        else:
                            pltpu.sync_copy(rows[b], acc.at[dstb.at[js[b]]],
                                            add=True)

                plsc.subcore_barrier()

            with jax.named_scope("writeout_phase"):
                pltpu.sync_copy(acc.at[pl.ds(rb, RPT)],
                                out_acc[r].at[cid].at[pl.ds(rb, RPT)])
                if with_deg:
                    @pl.when(sid == 0)
                    def _out_deg():
                        pltpu.sync_copy(deg, out_deg[r].at[cid])

    return run


@functools.lru_cache(maxsize=None)
def _edge_pass(width, ntab, with_deg, tab_bf16=False):
    return _make_edge_pass(width, ntab, with_deg, tab_bf16)


def _h1y_body(a0, a1, a2, dg0, dg1, dg2, b1, w2, y0, y1, y2):
    accs = (a0, a1, a2)
    dgs = (dg0, dg1, dg2)
    h = jnp.zeros((BLK, H), jnp.float32)
    for r in range(R):
        rec = 1.0 / jnp.maximum(dgs[r][0] + dgs[r][1], 1.0)
        h = h + (accs[r][0] + accs[r][1]) * rec
    h1 = jnp.maximum(h + b1[...][None, :], 0.0)
    for r, y in enumerate((y0, y1, y2)):
        y[...] = jnp.dot(h1, w2[r], preferred_element_type=jnp.float32)


def _tc_h1_y(acc3, deg3, b1, w2):
    acc_spec = pl.BlockSpec((NC, BLK, H), lambda i: (0, i, 0))
    deg_spec = pl.BlockSpec((NC, BLK, 1), lambda i: (0, i, 0))
    return pl.pallas_call(
        _h1y_body,
        grid=(GRID,),
        in_specs=[acc_spec] * 3 + [deg_spec] * 3
        + [pl.BlockSpec((H,), lambda i: (0,)),
           pl.BlockSpec((R, H, OUT), lambda i: (0, 0, 0))],
        out_specs=[pl.BlockSpec((BLK, OUT), lambda i: (i, 0))] * 3,
        out_shape=[jax.ShapeDtypeStruct((P, OUT), jnp.float32)] * 3,
    )(*acc3, *deg3, b1, w2)


def _out_body(a0, a1, a2, dg0, dg1, dg2, b2, o):
    accs = (a0, a1, a2)
    dgs = (dg0, dg1, dg2)
    h = jnp.zeros((BLK, OUT), jnp.float32)
    for r in range(R):
        rec = 1.0 / jnp.maximum(dgs[r][0] + dgs[r][1], 1.0)
        h = h + (accs[r][0] + accs[r][1]) * rec
    o[...] = h + b2[...][None, :]


def _tc_out(acc2, deg3, b2):
    acc_spec = pl.BlockSpec((NC, BLK, OUT), lambda i: (0, i, 0))
    deg_spec = pl.BlockSpec((NC, BLK, 1), lambda i: (0, i, 0))
    return pl.pallas_call(
        _out_body,
        grid=(GRID,),
        in_specs=[acc_spec] * 3 + [deg_spec] * 3
        + [pl.BlockSpec((OUT,), lambda i: (0,))],
        out_specs=pl.BlockSpec((BLK, OUT), lambda i: (i, 0)),
        out_shape=jax.ShapeDtypeStruct((P, OUT), jnp.float32),
    )(*acc2, *deg3, b2)


def kernel(embed, edge_index_r0, edge_index_r1, edge_index_r2,
           h_bias1, weight2, h_bias2):
    pad = EPAD - E
    srcs, dsts = [], []
    for e in (edge_index_r0, edge_index_r1, edge_index_r2):
        e = e.astype(jnp.int32)
        srcs.append(jnp.concatenate([e[0], jnp.zeros((pad,), jnp.int32)]))
        dsts.append(jnp.concatenate([e[1], jnp.full((pad,), N, jnp.int32)]))
    srcs = jnp.stack(srcs).reshape(R, NCHUNK, CH)
    dsts = jnp.stack(dsts).reshape(R, NCHUNK, CH)
    embed = embed.astype(jnp.float32)

    emb16 = (embed.astype(jnp.bfloat16)
             .reshape(N, H // 32, 2, L).swapaxes(2, 3).reshape(N, H))
    res = _edge_pass(H, 1, True, True)(emb16, srcs, dsts)
    acc1, deg3 = res[:R], res[R:]
    deg3 = [d.reshape(NC, P, 1) for d in deg3]

    ys = _tc_h1_y(acc1, deg3, h_bias1.astype(jnp.float32),
                  weight2.astype(jnp.float32))

    acc2 = _edge_pass(OUT, R, False)(ys[0], ys[1], ys[2], srcs, dsts)

    h2 = _tc_out(acc2, deg3, h_bias2.astype(jnp.float32))
    return h2[:N]


# speedup vs baseline: 1.3331x; 1.3331x over previous
import functools

import jax
import jax.numpy as jnp
from jax import lax
from jax.experimental import pallas as pl
from jax.experimental.pallas import tpu as pltpu
from jax.experimental.pallas import tpu_sc as plsc

N = 10000
H = 128
OUT = 64
R = 3
E = 320000

NC = 2
NS = 16
L = 16

CH = 128
NBUF = 2
S0 = 152
S1 = 8
NCHUNK = NS * (S0 + S1)
EPAD = NCHUNK * CH
IB = 20
QPB = IB // NBUF

RPT = 632
P = NS * RPT
ZR = 16

BLK = 1264
GRID = P // BLK


def _make_edge_pass(width, ntab, with_deg, tab_bf16=False):
    mesh = plsc.VectorSubcoreMesh(core_axis_name="c", subcore_axis_name="s")
    n_deg = R if with_deg else 0
    row_dt = jnp.bfloat16 if tab_bf16 else jnp.float32
    out_type = (
        [jax.ShapeDtypeStruct((NC, P, width), jnp.float32) for _ in range(R)]
        + [jax.ShapeDtypeStruct((NC, P), jnp.float32) for _ in range(n_deg)]
    )
    scratch = (
        [pltpu.VMEM_SHARED((P, width), jnp.float32),
         pltpu.VMEM((IB, CH), jnp.int32),
         pltpu.VMEM((IB, CH), jnp.int32)]
        + [pltpu.VMEM((CH, width), row_dt) for _ in range(NBUF)]
        + ([pltpu.VMEM((CH, width), jnp.float32)] if tab_bf16 else [])
        + [pltpu.VMEM((ZR, width), jnp.float32)]
        + [pltpu.SemaphoreType.DMA for _ in range(NBUF)]
    )
    if with_deg:
        scratch.insert(1, pltpu.VMEM_SHARED((P,), jnp.float32))
        scratch.append(pltpu.VMEM((CH,), jnp.float32))
        scratch.append(pltpu.VMEM((CH,), jnp.float32))

    @functools.partial(
        pl.kernel, out_type=out_type, scratch_types=scratch, mesh=mesh,
        name="edge_pass",
        compiler_params=pltpu.CompilerParams(
            use_tc_tiling_on_sc=False,
            needs_layout_passes=not tab_bf16))
    def run(*refs):
        i = 0
        tabs = refs[i:i + ntab]; i += ntab
        srcs_hbm, dsts_hbm = refs[i:i + 2]; i += 2
        out_acc = refs[i:i + R]; i += R
        out_deg = refs[i:i + n_deg]; i += n_deg
        acc = refs[i]; i += 1
        if with_deg:
            deg = refs[i]; i += 1
        srcb, dstb = refs[i:i + 2]; i += 2
        rows = refs[i:i + NBUF]; i += NBUF
        if tab_bf16:
            fbuf = refs[i]; i += 1
        zbuf = refs[i]; i += 1
        sems = refs[i:i + NBUF]; i += NBUF
        if with_deg:
            ones_v, z1v = refs[i:i + 2]

        cid = lax.axis_index("c")
        sid = lax.axis_index("s")
        rb = sid * RPT
        cbase = jnp.where(cid == 0, sid * S0, NS * S0 + sid * S1)
        nquads = jnp.where(cid == 0, S0 // NBUF, S1 // NBUF)

        zv = jnp.zeros((L,), jnp.float32)
        for zi in range(ZR):
            for zk in range(width // L):
                zbuf[zi, pl.ds(zk * L, L)] = zv
        if with_deg:
            ov = jnp.full((L,), 1.0, jnp.float32)
            for zk in range(CH // L):
                ones_v[pl.ds(zk * L, L)] = ov
            for zk in range(CH // L):
                z1v[pl.ds(zk * L, L)] = zv

        for r in range(R):
            with jax.named_scope("zero_phase"):
                @pl.loop(0, RPT // ZR)
                def _zero(k):
                    pltpu.sync_copy(zbuf, acc.at[pl.ds(rb + k * ZR, ZR)])
                if RPT % ZR:
                    pltpu.sync_copy(
                        zbuf.at[pl.ds(0, RPT % ZR)],
                        acc.at[pl.ds(rb + RPT - RPT % ZR, RPT % ZR)])
                if with_deg:
                    @pl.when(sid == 0)
                    def _zero_deg():
                        @pl.loop(0, P // CH)
                        def _zd(k):
                            pltpu.sync_copy(z1v, deg.at[pl.ds(k * CH, CH)])
                plsc.subcore_barrier()

            tab = tabs[r % ntab]

            with jax.named_scope("edges_phase"):
                @pl.loop(0, nquads)
                def _quad(q, tab=tab):
                    @pl.when(q % QPB == 0)
                    def _reload():
                        blk = pl.ds(cbase + NBUF * q, IB)
                        pltpu.sync_copy(srcs_hbm.at[r].at[blk], srcb)
                        pltpu.sync_copy(dsts_hbm.at[r].at[blk], dstb)
                    js = [(NBUF * q + b) % IB for b in range(NBUF)]
                    cps = [pltpu.async_copy(tab.at[srcb.at[js[b]]],
                                            rows[b], sems[b])
                           for b in range(NBUF)]
                    if with_deg:
                        for b in range(NBUF):
                            pltpu.sync_copy(ones_v, deg.at[dstb.at[js[b]]],
                                            add=True)
                    for b in range(NBUF):
                        cps[b].wait()
                        if tab_bf16:
                            @pl.loop(0, CH)
                            def _cvt(row, b=b):
                                for g in range(width // 32):
                                    v = rows[b][row, pl.ds(g * 32, 32)]
                                    lo, hi = plsc.unpack(
                                        v, format=plsc.PackFormat.INTERLEAVED)
                                    fbuf[row, pl.ds(g * 32, L)] = lo
                                    fbuf[row, pl.ds(g * 32 + L, L)] = hi
                            pltpu.sync_copy(fbuf, acc.at[dstb.at[js[b]]],
                                            add=True)
                        else:
                            pltpu.sync_copy(rows[b], acc.at[dstb.at[js[b]]],
                                            add=True)

                plsc.subcore_barrier()

            with jax.named_scope("writeout_phase"):
                pltpu.sync_copy(acc.at[pl.ds(rb, RPT)],
                                out_acc[r].at[cid].at[pl.ds(rb, RPT)])
                if with_deg:
                    @pl.when(sid == 0)
                    def _out_deg():
                        pltpu.sync_copy(deg, out_deg[r].at[cid])

    return run


@functools.lru_cache(maxsize=None)
def _edge_pass(width, ntab, with_deg, tab_bf16=False):
    return _make_edge_pass(width, ntab, with_deg, tab_bf16)


def _h1y_body(a0, a1, a2, dg0, dg1, dg2, b1, w2, y0, y1, y2):
    accs = (a0, a1, a2)
    dgs = (dg0, dg1, dg2)
    h = jnp.zeros((BLK, H), jnp.float32)
    for r in range(R):
        rec = 1.0 / jnp.maximum(dgs[r][0] + dgs[r][1], 1.0)
        h = h + (accs[r][0] + accs[r][1]) * rec
    h1 = jnp.maximum(h + b1[...][None, :], 0.0)
    for r, y in enumerate((y0, y1, y2)):
        y[...] = jnp.dot(h1, w2[r], preferred_element_type=jnp.float32)


def _tc_h1_y(acc3, deg3, b1, w2):
    acc_spec = pl.BlockSpec((NC, BLK, H), lambda i: (0, i, 0))
    deg_spec = pl.BlockSpec((NC, BLK, 1), lambda i: (0, i, 0))
    return pl.pallas_call(
        _h1y_body,
        grid=(GRID,),
        in_specs=[acc_spec] * 3 + [deg_spec] * 3
        + [pl.BlockSpec((H,), lambda i: (0,)),
           pl.BlockSpec((R, H, OUT), lambda i: (0, 0, 0))],
        out_specs=[pl.BlockSpec((BLK, OUT), lambda i: (i, 0))] * 3,
        out_shape=[jax.ShapeDtypeStruct((P, OUT), jnp.float32)] * 3,
    )(*acc3, *deg3, b1, w2)


def _out_body(a0, a1, a2, dg0, dg1, dg2, b2, o):
    accs = (a0, a1, a2)
    dgs = (dg0, dg1, dg2)
    h = jnp.zeros((BLK, OUT), jnp.float32)
    for r in range(R):
        rec = 1.0 / jnp.maximum(dgs[r][0] + dgs[r][1], 1.0)
        h = h + (accs[r][0] + accs[r][1]) * rec
    o[...] = h + b2[...][None, :]


def _tc_out(acc2, deg3, b2):
    acc_spec = pl.BlockSpec((NC, BLK, OUT), lambda i: (0, i, 0))
    deg_spec = pl.BlockSpec((NC, BLK, 1), lambda i: (0, i, 0))
    return pl.pallas_call(
        _out_body,
        grid=(GRID,),
        in_specs=[acc_spec] * 3 + [deg_spec] * 3
        + [pl.BlockSpec((OUT,), lambda i: (0,))],
        out_specs=pl.BlockSpec((BLK, OUT), lambda i: (i, 0)),
        out_shape=jax.ShapeDtypeStruct((P, OUT), jnp.float32),
    )(*acc2, *deg3, b2)


def kernel(embed, edge_index_r0, edge_index_r1, edge_index_r2,
           h_bias1, weight2, h_bias2):
    pad = EPAD - E
    srcs, dsts = [], []
    for e in (edge_index_r0, edge_index_r1, edge_index_r2):
        e = e.astype(jnp.int32)
        srcs.append(jnp.concatenate([e[0], jnp.zeros((pad,), jnp.int32)]))
        dsts.append(jnp.concatenate([e[1], jnp.full((pad,), N, jnp.int32)]))
    srcs = jnp.stack(srcs).reshape(R, NCHUNK, CH)
    dsts = jnp.stack(dsts).reshape(R, NCHUNK, CH)
    embed = embed.astype(jnp.float32)

    res = _edge_pass(H, 1, True)(embed, srcs, dsts)
    acc1, deg3 = res[:R], res[R:]
    deg3 = [d.reshape(NC, P, 1) for d in deg3]

    ys = _tc_h1_y(acc1, deg3, h_bias1.astype(jnp.float32),
                  weight2.astype(jnp.float32))

    acc2 = _edge_pass(OUT, R, False)(ys[0], ys[1], ys[2], srcs, dsts)

    h2 = _tc_out(acc2, deg3, h_bias2.astype(jnp.float32))
    return h2[:N]


# speedup vs baseline: 1.3849x; 1.0388x over previous
import functools

import jax
import jax.numpy as jnp
from jax import lax
from jax.experimental import pallas as pl
from jax.experimental.pallas import tpu as pltpu
from jax.experimental.pallas import tpu_sc as plsc

N = 10000
H = 128
OUT = 64
R = 3
E = 320000

NC = 2
NS = 16
L = 16

CH = 128
NBUF = 2
S0 = 146
S1 = 14
NCHUNK = NS * (S0 + S1)
EPAD = NCHUNK * CH
IB = 20
QPB = IB // NBUF

RPT = 632
P = NS * RPT
ZR = 16

BLK = 1264
GRID = P // BLK


def _make_edge_pass(width, ntab, with_deg, tab_bf16=False):
    mesh = plsc.VectorSubcoreMesh(core_axis_name="c", subcore_axis_name="s")
    n_deg = R if with_deg else 0
    row_dt = jnp.bfloat16 if tab_bf16 else jnp.float32
    out_type = (
        [jax.ShapeDtypeStruct((NC, P, width), jnp.float32) for _ in range(R)]
        + [jax.ShapeDtypeStruct((NC, P), jnp.float32) for _ in range(n_deg)]
    )
    scratch = (
        [pltpu.VMEM_SHARED((P, width), jnp.float32),
         pltpu.VMEM((IB, CH), jnp.int32),
         pltpu.VMEM((IB, CH), jnp.int32)]
        + [pltpu.VMEM((CH, width), row_dt) for _ in range(NBUF)]
        + ([pltpu.VMEM((CH, width), jnp.float32)] if tab_bf16 else [])
        + [pltpu.VMEM((ZR, width), jnp.float32)]
        + [pltpu.SemaphoreType.DMA for _ in range(NBUF)]
    )
    if with_deg:
        scratch.insert(1, pltpu.VMEM_SHARED((P,), jnp.float32))
        scratch.append(pltpu.VMEM((CH,), jnp.float32))
        scratch.append(pltpu.VMEM((CH,), jnp.float32))

    @functools.partial(
        pl.kernel, out_type=out_type, scratch_types=scratch, mesh=mesh,
        name="edge_pass",
        compiler_params=pltpu.CompilerParams(
            use_tc_tiling_on_sc=False,
            needs_layout_passes=not tab_bf16))
    def run(*refs):
        i = 0
        tabs = refs[i:i + ntab]; i += ntab
        srcs_hbm, dsts_hbm = refs[i:i + 2]; i += 2
        out_acc = refs[i:i + R]; i += R
        out_deg = refs[i:i + n_deg]; i += n_deg
        acc = refs[i]; i += 1
        if with_deg:
            deg = refs[i]; i += 1
        srcb, dstb = refs[i:i + 2]; i += 2
        rows = refs[i:i + NBUF]; i += NBUF
        if tab_bf16:
            fbuf = refs[i]; i += 1
        zbuf = refs[i]; i += 1
        sems = refs[i:i + NBUF]; i += NBUF
        if with_deg:
            ones_v, z1v = refs[i:i + 2]

        cid = lax.axis_index("c")
        sid = lax.axis_index("s")
        rb = sid * RPT
        cbase = jnp.where(cid == 0, sid * S0, NS * S0 + sid * S1)
        nquads = jnp.where(cid == 0, S0 // NBUF, S1 // NBUF)

        zv = jnp.zeros((L,), jnp.float32)
        for zi in range(ZR):
            for zk in range(width // L):
                zbuf[zi, pl.ds(zk * L, L)] = zv
        if with_deg:
            ov = jnp.full((L,), 1.0, jnp.float32)
            for zk in range(CH // L):
                ones_v[pl.ds(zk * L, L)] = ov
            for zk in range(CH // L):
                z1v[pl.ds(zk * L, L)] = zv

        for r in range(R):
            with jax.named_scope("zero_phase"):
                @pl.loop(0, RPT // ZR)
                def _zero(k):
                    pltpu.sync_copy(zbuf, acc.at[pl.ds(rb + k * ZR, ZR)])
                if RPT % ZR:
                    pltpu.sync_copy(
                        zbuf.at[pl.ds(0, RPT % ZR)],
                        acc.at[pl.ds(rb + RPT - RPT % ZR, RPT % ZR)])
                if with_deg:
                    @pl.when(sid == 0)
                    def _zero_deg():
                        @pl.loop(0, P // CH)
                        def _zd(k):
                            pltpu.sync_copy(z1v, deg.at[pl.ds(k * CH, CH)])
                plsc.subcore_barrier()

            tab = tabs[r % ntab]

            with jax.named_scope("edges_phase"):
                @pl.loop(0, nquads)
                def _quad(q, tab=tab):
                    @pl.when(q % QPB == 0)
                    def _reload():
                        blk = pl.ds(cbase + NBUF * q, IB)
                        pltpu.sync_copy(srcs_hbm.at[r].at[blk], srcb)
                        pltpu.sync_copy(dsts_hbm.at[r].at[blk], dstb)
                    js = [(NBUF * q + b) % IB for b in range(NBUF)]
                    cps = [pltpu.async_copy(tab.at[srcb.at[js[b]]],
                                            rows[b], sems[b])
                           for b in range(NBUF)]
                    if with_deg:
                        for b in range(NBUF):
                            pltpu.sync_copy(ones_v, deg.at[dstb.at[js[b]]],
                                            add=True)
                    for b in range(NBUF):
                        cps[b].wait()
                        if tab_bf16:
                            @pl.loop(0, CH)
                            def _cvt(row, b=b):
                                for g in range(width // 32):
                                    v = rows[b][row, pl.ds(g * 32, 32)]
                                    lo, hi = plsc.unpack(
                                        v, format=plsc.PackFormat.INTERLEAVED)
                                    fbuf[row, pl.ds(g * 32, L)] = lo
                                    fbuf[row, pl.ds(g * 32 + L, L)] = hi
                            pltpu.sync_copy(fbuf, acc.at[dstb.at[js[b]]],
                                            add=True)
                        else:
                            pltpu.sync_copy(rows[b], acc.at[dstb.at[js[b]]],
                                            add=True)

                plsc.subcore_barrier()

            with jax.named_scope("writeout_phase"):
                pltpu.sync_copy(acc.at[pl.ds(rb, RPT)],
                                out_acc[r].at[cid].at[pl.ds(rb, RPT)])
                if with_deg:
                    @pl.when(sid == 0)
                    def _out_deg():
                        pltpu.sync_copy(deg, out_deg[r].at[cid])

    return run


@functools.lru_cache(maxsize=None)
def _edge_pass(width, ntab, with_deg, tab_bf16=False):
    return _make_edge_pass(width, ntab, with_deg, tab_bf16)


def _h1y_body(a0, a1, a2, dg0, dg1, dg2, b1, w2, y0, y1, y2):
    accs = (a0, a1, a2)
    dgs = (dg0, dg1, dg2)
    h = jnp.zeros((BLK, H), jnp.float32)
    for r in range(R):
        rec = 1.0 / jnp.maximum(dgs[r][0] + dgs[r][1], 1.0)
        h = h + (accs[r][0] + accs[r][1]) * rec
    h1 = jnp.maximum(h + b1[...][None, :], 0.0)
    for r, y in enumerate((y0, y1, y2)):
        y[...] = jnp.dot(h1, w2[r], preferred_element_type=jnp.float32)


def _tc_h1_y(acc3, deg3, b1, w2):
    acc_spec = pl.BlockSpec((NC, BLK, H), lambda i: (0, i, 0))
    deg_spec = pl.BlockSpec((NC, BLK, 1), lambda i: (0, i, 0))
    return pl.pallas_call(
        _h1y_body,
        grid=(GRID,),
        in_specs=[acc_spec] * 3 + [deg_spec] * 3
        + [pl.BlockSpec((H,), lambda i: (0,)),
           pl.BlockSpec((R, H, OUT), lambda i: (0, 0, 0))],
        out_specs=[pl.BlockSpec((BLK, OUT), lambda i: (i, 0))] * 3,
        out_shape=[jax.ShapeDtypeStruct((P, OUT), jnp.float32)] * 3,
    )(*acc3, *deg3, b1, w2)


def _out_body(a0, a1, a2, dg0, dg1, dg2, b2, o):
    accs = (a0, a1, a2)
    dgs = (dg0, dg1, dg2)
    h = jnp.zeros((BLK, OUT), jnp.float32)
    for r in range(R):
        rec = 1.0 / jnp.maximum(dgs[r][0] + dgs[r][1], 1.0)
        h = h + (accs[r][0] + accs[r][1]) * rec
    o[...] = h + b2[...][None, :]


def _tc_out(acc2, deg3, b2):
    acc_spec = pl.BlockSpec((NC, BLK, OUT), lambda i: (0, i, 0))
    deg_spec = pl.BlockSpec((NC, BLK, 1), lambda i: (0, i, 0))
    return pl.pallas_call(
        _out_body,
        grid=(GRID,),
        in_specs=[acc_spec] * 3 + [deg_spec] * 3
        + [pl.BlockSpec((OUT,), lambda i: (0,))],
        out_specs=pl.BlockSpec((BLK, OUT), lambda i: (i, 0)),
        out_shape=jax.ShapeDtypeStruct((P, OUT), jnp.float32),
    )(*acc2, *deg3, b2)


def kernel(embed, edge_index_r0, edge_index_r1, edge_index_r2,
           h_bias1, weight2, h_bias2):
    pad = EPAD - E
    srcs, dsts = [], []
    for e in (edge_index_r0, edge_index_r1, edge_index_r2):
        e = e.astype(jnp.int32)
        srcs.append(jnp.concatenate([e[0], jnp.zeros((pad,), jnp.int32)]))
        dsts.append(jnp.concatenate([e[1], jnp.full((pad,), N, jnp.int32)]))
    srcs = jnp.stack(srcs).reshape(R, NCHUNK, CH)
    dsts = jnp.stack(dsts).reshape(R, NCHUNK, CH)
    embed = embed.astype(jnp.float32)

    res = _edge_pass(H, 1, True)(embed, srcs, dsts)
    acc1, deg3 = res[:R], res[R:]
    deg3 = [d.reshape(NC, P, 1) for d in deg3]

    ys = _tc_h1_y(acc1, deg3, h_bias1.astype(jnp.float32),
                  weight2.astype(jnp.float32))

    acc2 = _edge_pass(OUT, R, False)(ys[0], ys[1], ys[2], srcs, dsts)

    h2 = _tc_out(acc2, deg3, h_bias2.astype(jnp.float32))
    return h2[:N]
